# baseline (speedup 1.0000x reference)
import functools

import jax
import jax.numpy as jnp
from jax import lax
from jax.experimental import pallas as pl
from jax.experimental.pallas import tpu as pltpu
from jax.experimental.pallas import tpu_sc as plsc

NC = 2
NS = 16
NW = NC * NS
L = 16
CH = 128


def _ceil_to(a: int, m: int) -> int:
    return (a + m - 1) // m * m


def _deg_kernel(N, Np, EPW):
    mesh = plsc.VectorSubcoreMesh(core_axis_name="c", subcore_axis_name="s")

    @functools.partial(
        pl.kernel,
        out_type=jax.ShapeDtypeStruct((NW, Np), jnp.float32),
        mesh=mesh,
        scratch_types=[
            pltpu.VMEM((EPW,), jnp.int32),
            pltpu.VMEM((Np,), jnp.float32),
        ],
        compiler_params=pltpu.CompilerParams(needs_layout_passes=False),
    )
    def deg_k(dst_hbm, zero_hbm, out_hbm, dstv, degv):
        cid = lax.axis_index("c")
        sid = lax.axis_index("s")
        wid = sid * NC + cid
        pltpu.sync_copy(dst_hbm.at[wid], dstv)
        pltpu.sync_copy(zero_hbm, degv)
        ones = jnp.ones((L,), jnp.float32)

        for j in range(EPW // L):
            idx = dstv[pl.ds(j * L, L)]
            plsc.addupdate_scatter(degv, [idx], ones)

        pltpu.sync_copy(degv, out_hbm.at[wid])

    return deg_k


def _scatter_kernel(N, OUT, Np, NJ, G):
    mesh = plsc.VectorSubcoreMesh(core_axis_name="c", subcore_axis_name="s")
    rpt = Np // NS
    NG = NJ // G

    @functools.partial(
        pl.kernel,
        out_type=jax.ShapeDtypeStruct((NC, Np, OUT), jnp.float32),
        mesh=mesh,
        scratch_types=[
            pltpu.VMEM((NJ, CH), jnp.int32),
            pltpu.VMEM((NJ, CH), jnp.int32),
            pltpu.VMEM((2, G, CH, OUT), jnp.float32),
            pltpu.VMEM_SHARED((Np, OUT), jnp.float32),
            pltpu.VMEM_SHARED((Np, OUT), jnp.float32),
            pltpu.SemaphoreType.DMA,
            pltpu.SemaphoreType.DMA,
        ],
        compiler_params=pltpu.CompilerParams(use_tc_tiling_on_sc=False),
    )
    def sc2(g_hbm, src_hbm, dst_hbm, zero_hbm, out_hbm, srcv, dstv, rows, acc,
            g_sp, gsem, ssem):
        cid = lax.axis_index("c")
        sid = lax.axis_index("s")
        wid = sid * NC + cid
        pltpu.sync_copy(src_hbm.at[wid], srcv)
        pltpu.sync_copy(dst_hbm.at[wid], dstv)
        pltpu.sync_copy(
            zero_hbm.at[pl.ds(sid * rpt, rpt)], acc.at[pl.ds(sid * rpt, rpt)]
        )
        pltpu.sync_copy(
            g_hbm.at[pl.ds(sid * rpt, rpt)], g_sp.at[pl.ds(sid * rpt, rpt)]
        )
        plsc.subcore_barrier()

        def _fire_gathers(n, bank):
            for i in range(G):
                j = n * G + i
                pltpu.async_copy(g_sp.at[srcv.at[j]], rows.at[bank, i], gsem)

        def _fire_scatters(n, bank):
            for i in range(G):
                j = n * G + i
                pltpu.async_copy(
                    rows.at[bank, i], acc.at[dstv.at[j]], ssem, add=True
                )

        def _drain(sem, k):
            for _ in range(k):
                pltpu.make_async_copy(
                    g_hbm.at[srcv.at[0]], rows.at[0, 0], sem
                ).wait()

        for n in range(2):
            _fire_gathers(n, n)
            _drain(gsem, G)
            _fire_scatters(n, n)

        for n in range(2, NG):
            bank = n % 2
            _drain(ssem, G)
            _fire_gathers(n, bank)
            _drain(gsem, G)
            _fire_scatters(n, bank)
        _drain(ssem, 2 * G)
        plsc.subcore_barrier()
        pltpu.sync_copy(
            acc.at[pl.ds(sid * rpt, rpt)], out_hbm.at[cid, pl.ds(sid * rpt, rpt)]
        )

    return sc2


def _tc_matmul(N, IN, OUT):
    def body(x_ref, w_ref, h_ref):
        h_ref[...] = lax.dot_general(
            x_ref[...], w_ref[...], (((1,), (1,)), ((), ())),
            preferred_element_type=jnp.float32,
        )

    return pl.pallas_call(
        body, out_shape=jax.ShapeDtypeStruct((N, OUT), jnp.float32)
    )


def _tc_scale(N, OUT):
    def body(h_ref, degp_ref, g_ref, dinv_ref):
        deg = jnp.sum(degp_ref[:, :N], axis=0) + 1.0
        dinv = lax.rsqrt(deg)
        g_ref[...] = h_ref[...] * dinv[:, None]
        dinv_ref[...] = dinv

    return pl.pallas_call(
        body,
        out_shape=[
            jax.ShapeDtypeStruct((N, OUT), jnp.float32),
            jax.ShapeDtypeStruct((N,), jnp.float32),
        ],
    )


def _tc_final(N, OUT):
    def body(accp_ref, g_ref, dinv_ref, b_ref, o_ref):
        s = jnp.sum(accp_ref[:, :N, :], axis=0) + g_ref[...]
        o_ref[...] = jnp.maximum(
            s * dinv_ref[...][:, None] + b_ref[...][None, :], 0.0
        )

    return pl.pallas_call(
        body, out_shape=jax.ShapeDtypeStruct((N, OUT), jnp.float32)
    )


def kernel(x, edge_index, W, b):
    N, IN = x.shape
    OUT = W.shape[0]
    E = edge_index.shape[1]

    G = 4
    NJ = _ceil_to(-(-E // (NW * CH)), 2 * G)
    EPW = NJ * CH
    Epad = NW * EPW
    Np = _ceil_to(N + 1, 8 * NS)

    padn = Epad - E
    src = jnp.concatenate(
        [edge_index[0], jnp.zeros((padn,), jnp.int32)]
    )
    dst = jnp.concatenate(
        [edge_index[1], jnp.full((padn,), N, jnp.int32)]
    )
    src3 = src.reshape(NW, NJ, CH)
    dst3 = dst.reshape(NW, NJ, CH)
    dst2 = dst.reshape(NW, EPW)
    zeros1 = jnp.zeros((Np,), jnp.float32)
    zeros2 = jnp.zeros((Np, OUT), jnp.float32)

    degP = _deg_kernel(N, Np, EPW)(dst2, zeros1)
    h = _tc_matmul(N, IN, OUT)(x, W)
    g, dinv = _tc_scale(N, OUT)(h, degP)
    accP = _scatter_kernel(N, OUT, Np, NJ, G)(g, src3, dst3, zeros2)
    return _tc_final(N, OUT)(accP, g, dinv, b)


# speedup vs baseline: 1.0320x; 1.0320x over previous
import functools

import jax
import jax.numpy as jnp
from jax import lax
from jax.experimental import pallas as pl
from jax.experimental.pallas import tpu as pltpu
from jax.experimental.pallas import tpu_sc as plsc

NC = 2
NS = 16
NW = NC * NS
L = 16
CH = 128


def _ceil_to(a: int, m: int) -> int:
    return (a + m - 1) // m * m


def _deg_kernel(N, Np, EPW):
    mesh = plsc.VectorSubcoreMesh(core_axis_name="c", subcore_axis_name="s")

    @functools.partial(
        pl.kernel,
        out_type=jax.ShapeDtypeStruct((NW, Np), jnp.float32),
        mesh=mesh,
        scratch_types=[
            pltpu.VMEM((EPW,), jnp.int32),
            pltpu.VMEM((Np,), jnp.float32),
        ],
        compiler_params=pltpu.CompilerParams(needs_layout_passes=False),
    )
    def deg_k(dst_hbm, zero_hbm, out_hbm, dstv, degv):
        cid = lax.axis_index("c")
        sid = lax.axis_index("s")
        wid = sid * NC + cid
        pltpu.sync_copy(dst_hbm.at[wid], dstv)
        pltpu.sync_copy(zero_hbm, degv)
        ones = jnp.ones((L,), jnp.float32)

        def body(j, carry):
            for i in range(8):
                idx = dstv[pl.ds((j * 8 + i) * L, L)]
                plsc.addupdate_scatter(degv, [idx], ones)
            return carry

        lax.fori_loop(0, EPW // (8 * L), body, 0)
        pltpu.sync_copy(degv, out_hbm.at[wid])

    return deg_k


def _scatter_kernel(N, OUT, Np, NJ, G):
    mesh = plsc.VectorSubcoreMesh(core_axis_name="c", subcore_axis_name="s")
    rpt = Np // NS
    NG = NJ // G

    @functools.partial(
        pl.kernel,
        out_type=jax.ShapeDtypeStruct((NC, Np, OUT), jnp.float32),
        mesh=mesh,
        scratch_types=[
            pltpu.VMEM((NJ, CH), jnp.int32),
            pltpu.VMEM((NJ, CH), jnp.int32),
            pltpu.VMEM((2, G, CH, OUT), jnp.float32),
            pltpu.VMEM_SHARED((Np, OUT), jnp.float32),
            pltpu.VMEM_SHARED((Np, OUT), jnp.float32),
            pltpu.SemaphoreType.DMA,
            pltpu.SemaphoreType.DMA,
        ],
        compiler_params=pltpu.CompilerParams(use_tc_tiling_on_sc=False),
    )
    def sc2(g_hbm, src_hbm, dst_hbm, zero_hbm, out_hbm, srcv, dstv, rows, acc,
            g_sp, gsem, ssem):
        cid = lax.axis_index("c")
        sid = lax.axis_index("s")
        wid = sid * NC + cid
        pltpu.sync_copy(src_hbm.at[wid], srcv)
        pltpu.sync_copy(dst_hbm.at[wid], dstv)
        pltpu.sync_copy(
            zero_hbm.at[pl.ds(sid * rpt, rpt)], acc.at[pl.ds(sid * rpt, rpt)]
        )
        pltpu.sync_copy(
            g_hbm.at[pl.ds(sid * rpt, rpt)], g_sp.at[pl.ds(sid * rpt, rpt)]
        )
        plsc.subcore_barrier()

        def _fire_gathers(n, bank):
            for i in range(G):
                j = n * G + i
                pltpu.async_copy(g_sp.at[srcv.at[j]], rows.at[bank, i], gsem)

        def _fire_scatters(n, bank):
            for i in range(G):
                j = n * G + i
                pltpu.async_copy(
                    rows.at[bank, i], acc.at[dstv.at[j]], ssem, add=True
                )

        def _drain(sem, k):
            for _ in range(k):
                pltpu.make_async_copy(
                    g_hbm.at[srcv.at[0]], rows.at[0, 0], sem
                ).wait()

        for n in range(2):
            _fire_gathers(n, n)
            _drain(gsem, G)
            _fire_scatters(n, n)

        def body(n, carry):
            bank = n % 2
            _drain(ssem, G)
            _fire_gathers(n, bank)
            _drain(gsem, G)
            _fire_scatters(n, bank)
            return carry

        lax.fori_loop(2, NG, body, 0)
        _drain(ssem, 2 * G)
        plsc.subcore_barrier()
        pltpu.sync_copy(
            acc.at[pl.ds(sid * rpt, rpt)], out_hbm.at[cid, pl.ds(sid * rpt, rpt)]
        )

    return sc2


def _tc_matmul(N, IN, OUT):
    def body(x_ref, w_ref, h_ref):
        h_ref[...] = lax.dot_general(
            x_ref[...], w_ref[...], (((1,), (1,)), ((), ())),
            preferred_element_type=jnp.float32,
        )

    return pl.pallas_call(
        body, out_shape=jax.ShapeDtypeStruct((N, OUT), jnp.float32)
    )


def _tc_scale(N, OUT):
    def body(h_ref, degp_ref, g_ref, dinv_ref):
        deg = jnp.sum(degp_ref[:, :N], axis=0) + 1.0
        dinv = lax.rsqrt(deg)
        g_ref[...] = h_ref[...] * dinv[:, None]
        dinv_ref[...] = dinv

    return pl.pallas_call(
        body,
        out_shape=[
            jax.ShapeDtypeStruct((N, OUT), jnp.float32),
            jax.ShapeDtypeStruct((N,), jnp.float32),
        ],
    )


def _tc_final(N, OUT):
    def body(accp_ref, g_ref, dinv_ref, b_ref, o_ref):
        s = jnp.sum(accp_ref[:, :N, :], axis=0) + g_ref[...]
        o_ref[...] = jnp.maximum(
            s * dinv_ref[...][:, None] + b_ref[...][None, :], 0.0
        )

    return pl.pallas_call(
        body, out_shape=jax.ShapeDtypeStruct((N, OUT), jnp.float32)
    )


def kernel(x, edge_index, W, b):
    N, IN = x.shape
    OUT = W.shape[0]
    E = edge_index.shape[1]

    G = 4
    NJ = _ceil_to(-(-E // (NW * CH)), 2 * G)
    EPW = NJ * CH
    Epad = NW * EPW
    Np = _ceil_to(N + 1, 8 * NS)

    padn = Epad - E
    src = jnp.concatenate(
        [edge_index[0], jnp.zeros((padn,), jnp.int32)]
    )
    dst = jnp.concatenate(
        [edge_index[1], jnp.full((padn,), N, jnp.int32)]
    )
    src3 = src.reshape(NW, NJ, CH)
    dst3 = dst.reshape(NW, NJ, CH)
    dst2 = dst.reshape(NW, EPW)
    zeros1 = jnp.zeros((Np,), jnp.float32)
    zeros2 = jnp.zeros((Np, OUT), jnp.float32)

    degP = _deg_kernel(N, Np, EPW)(dst2, zeros1)
    h = _tc_matmul(N, IN, OUT)(x, W)
    g, dinv = _tc_scale(N, OUT)(h, degP)
    accP = _scatter_kernel(N, OUT, Np, NJ, G)(g, src3, dst3, zeros2)
    return _tc_final(N, OUT)(accP, g, dinv, b)


# speedup vs baseline: 1.0695x; 1.0363x over previous
import functools

import jax
import jax.numpy as jnp
from jax import lax
from jax.experimental import pallas as pl
from jax.experimental.pallas import tpu as pltpu
from jax.experimental.pallas import tpu_sc as plsc

NC = 2
NS = 16
NW = NC * NS
L = 16
CH = 128


def _ceil_to(a: int, m: int) -> int:
    return (a + m - 1) // m * m


def _deg_kernel(N, Np, EPW):
    mesh = plsc.VectorSubcoreMesh(core_axis_name="c", subcore_axis_name="s")

    @functools.partial(
        pl.kernel,
        out_type=jax.ShapeDtypeStruct((NW, Np), jnp.float32),
        mesh=mesh,
        scratch_types=[
            pltpu.VMEM((EPW,), jnp.int32),
            pltpu.VMEM((Np,), jnp.float32),
        ],
        compiler_params=pltpu.CompilerParams(needs_layout_passes=False),
    )
    def deg_k(dst_hbm, out_hbm, dstv, degv):
        cid = lax.axis_index("c")
        sid = lax.axis_index("s")
        wid = sid * NC + cid
        pltpu.sync_copy(dst_hbm.at[wid], dstv)
        zl = jnp.zeros((L,), jnp.float32)

        def zbody(j, carry):
            for i in range(8):
                degv[pl.ds((j * 8 + i) * L, L)] = zl
            return carry

        lax.fori_loop(0, Np // (8 * L), zbody, 0)
        ones = jnp.ones((L,), jnp.float32)

        def body(j, carry):
            for i in range(8):
                idx = dstv[pl.ds((j * 8 + i) * L, L)]
                plsc.addupdate_scatter(degv, [idx], ones)
            return carry

        lax.fori_loop(0, EPW // (8 * L), body, 0)
        pltpu.sync_copy(degv, out_hbm.at[wid])

    return deg_k


def _scatter_kernel(N, OUT, Np, NJ, G):
    mesh = plsc.VectorSubcoreMesh(core_axis_name="c", subcore_axis_name="s")
    rpt = Np // NS
    NG = NJ // G

    @functools.partial(
        pl.kernel,
        out_type=jax.ShapeDtypeStruct((NC, Np, OUT), jnp.float32),
        mesh=mesh,
        scratch_types=[
            pltpu.VMEM((NJ, CH), jnp.int32),
            pltpu.VMEM((NJ, CH), jnp.int32),
            pltpu.VMEM((2, G, CH, OUT), jnp.float32),
            pltpu.VMEM((rpt, OUT), jnp.float32),
            pltpu.VMEM_SHARED((Np, OUT), jnp.float32),
            pltpu.VMEM_SHARED((Np, OUT), jnp.float32),
            pltpu.SemaphoreType.DMA,
            pltpu.SemaphoreType.DMA,
        ],
        compiler_params=pltpu.CompilerParams(use_tc_tiling_on_sc=False),
    )
    def sc2(g_hbm, src_hbm, dst_hbm, out_hbm, srcv, dstv, rows, zbuf, acc,
            g_sp, gsem, ssem):
        cid = lax.axis_index("c")
        sid = lax.axis_index("s")
        wid = sid * NC + cid
        pltpu.sync_copy(src_hbm.at[wid], srcv)
        pltpu.sync_copy(dst_hbm.at[wid], dstv)
        zrow = jnp.zeros((OUT,), jnp.float32)

        def zbody(j, carry):
            for i in range(8):
                zbuf[j * 8 + i, :] = zrow
            return carry

        lax.fori_loop(0, rpt // 8, zbody, 0)
        pltpu.sync_copy(zbuf, acc.at[pl.ds(sid * rpt, rpt)])
        pltpu.sync_copy(
            g_hbm.at[pl.ds(sid * rpt, rpt)], g_sp.at[pl.ds(sid * rpt, rpt)]
        )
        plsc.subcore_barrier()

        def _fire_gathers(n, bank):
            for i in range(G):
                j = n * G + i
                pltpu.async_copy(g_sp.at[srcv.at[j]], rows.at[bank, i], gsem)

        def _fire_scatters(n, bank):
            for i in range(G):
                j = n * G + i
                pltpu.async_copy(
                    rows.at[bank, i], acc.at[dstv.at[j]], ssem, add=True
                )

        def _drain(sem, k):
            for _ in range(k):
                pltpu.make_async_copy(
                    g_hbm.at[srcv.at[0]], rows.at[0, 0], sem
                ).wait()

        for n in range(2):
            _fire_gathers(n, n)
            _drain(gsem, G)
            _fire_scatters(n, n)

        def body(n, carry):
            bank = n % 2
            _drain(ssem, G)
            _fire_gathers(n, bank)
            _drain(gsem, G)
            _fire_scatters(n, bank)
            return carry

        lax.fori_loop(2, NG, body, 0)
        _drain(ssem, 2 * G)
        plsc.subcore_barrier()
        pltpu.sync_copy(
            acc.at[pl.ds(sid * rpt, rpt)], out_hbm.at[cid, pl.ds(sid * rpt, rpt)]
        )

    return sc2


def _tc_prep(N, IN, OUT):
    def body(x_ref, w_ref, degp_ref, g_ref, dinv_ref):
        h = lax.dot_general(
            x_ref[...], w_ref[...], (((1,), (1,)), ((), ())),
            preferred_element_type=jnp.float32,
        )
        deg = jnp.sum(degp_ref[:, :N], axis=0) + 1.0
        dinv = lax.rsqrt(deg)
        g_ref[...] = h * dinv[:, None]
        dinv_ref[...] = dinv

    return pl.pallas_call(
        body,
        out_shape=[
            jax.ShapeDtypeStruct((N, OUT), jnp.float32),
            jax.ShapeDtypeStruct((N,), jnp.float32),
        ],
    )


def _tc_final(N, OUT):
    def body(accp_ref, g_ref, dinv_ref, b_ref, o_ref):
        s = jnp.sum(accp_ref[:, :N, :], axis=0) + g_ref[...]
        o_ref[...] = jnp.maximum(
            s * dinv_ref[...][:, None] + b_ref[...][None, :], 0.0
        )

    return pl.pallas_call(
        body, out_shape=jax.ShapeDtypeStruct((N, OUT), jnp.float32)
    )


def kernel(x, edge_index, W, b):
    N, IN = x.shape
    OUT = W.shape[0]
    E = edge_index.shape[1]

    G = 4
    NJ = _ceil_to(-(-E // (NW * CH)), 2 * G)
    EPW = NJ * CH
    Epad = NW * EPW
    Np = _ceil_to(N + 1, 8 * NS)

    padn = Epad - E
    src = jnp.concatenate(
        [edge_index[0], jnp.zeros((padn,), jnp.int32)]
    )
    dst = jnp.concatenate(
        [edge_index[1], jnp.full((padn,), N, jnp.int32)]
    )
    src3 = src.reshape(NW, NJ, CH)
    dst3 = dst.reshape(NW, NJ, CH)
    dst2 = dst.reshape(NW, EPW)

    degP = _deg_kernel(N, Np, EPW)(dst2)
    g, dinv = _tc_prep(N, IN, OUT)(x, W, degP)
    accP = _scatter_kernel(N, OUT, Np, NJ, G)(g, src3, dst3)
    return _tc_final(N, OUT)(accP, g, dinv, b)


# speedup vs baseline: 1.0699x; 1.0004x over previous
import functools

import jax
import jax.numpy as jnp
from jax import lax
from jax.experimental import pallas as pl
from jax.experimental.pallas import tpu as pltpu
from jax.experimental.pallas import tpu_sc as plsc

NC = 2
NS = 16
NW = NC * NS
L = 16
CH = 128


def _ceil_to(a: int, m: int) -> int:
    return (a + m - 1) // m * m


def _deg_kernel(N, Np, EPW):
    mesh = plsc.VectorSubcoreMesh(core_axis_name="c", subcore_axis_name="s")

    @functools.partial(
        pl.kernel,
        out_type=jax.ShapeDtypeStruct((NW, Np), jnp.float32),
        mesh=mesh,
        scratch_types=[
            pltpu.VMEM((EPW,), jnp.int32),
            pltpu.VMEM((Np,), jnp.float32),
        ],
        compiler_params=pltpu.CompilerParams(needs_layout_passes=False),
    )
    def deg_k(dst_hbm, out_hbm, dstv, degv):
        cid = lax.axis_index("c")
        sid = lax.axis_index("s")
        wid = sid * NC + cid
        pltpu.sync_copy(dst_hbm.at[wid], dstv)
        zl = jnp.zeros((L,), jnp.float32)

        def zbody(j, carry):
            for i in range(8):
                degv[pl.ds((j * 8 + i) * L, L)] = zl
            return carry

        lax.fori_loop(0, Np // (8 * L), zbody, 0)
        ones = jnp.ones((L,), jnp.float32)

        def body(j, carry):
            for i in range(8):
                idx = dstv[pl.ds((j * 8 + i) * L, L)]
                plsc.addupdate_scatter(degv, [idx], ones)
            return carry

        lax.fori_loop(0, EPW // (8 * L), body, 0)
        pltpu.sync_copy(degv, out_hbm.at[wid])

    return deg_k


def _scatter_kernel(N, OUT, Np, NJ, G):
    mesh = plsc.VectorSubcoreMesh(core_axis_name="c", subcore_axis_name="s")
    rpt = Np // NS
    NG = NJ // G

    @functools.partial(
        pl.kernel,
        out_type=jax.ShapeDtypeStruct((NC, Np, OUT), jnp.float32),
        mesh=mesh,
        scratch_types=[
            pltpu.VMEM((NJ, CH), jnp.int32),
            pltpu.VMEM((NJ, CH), jnp.int32),
            pltpu.VMEM((2, G, CH, OUT), jnp.float32),
            pltpu.VMEM((rpt, OUT), jnp.float32),
            pltpu.VMEM_SHARED((Np, OUT), jnp.float32),
            pltpu.VMEM_SHARED((Np, OUT), jnp.float32),
            pltpu.SemaphoreType.DMA,
            pltpu.SemaphoreType.DMA,
        ],
        compiler_params=pltpu.CompilerParams(use_tc_tiling_on_sc=False),
    )
    def sc2(g_hbm, src_hbm, dst_hbm, out_hbm, srcv, dstv, rows, zbuf, acc,
            g_sp, gsem, ssem):
        cid = lax.axis_index("c")
        sid = lax.axis_index("s")
        wid = sid * NC + cid
        pltpu.sync_copy(src_hbm.at[wid], srcv)
        pltpu.sync_copy(dst_hbm.at[wid], dstv)
        zrow = jnp.zeros((OUT,), jnp.float32)

        def zbody(j, carry):
            for i in range(8):
                zbuf[j * 8 + i, :] = zrow
            return carry

        lax.fori_loop(0, rpt // 8, zbody, 0)
        pltpu.sync_copy(zbuf, acc.at[pl.ds(sid * rpt, rpt)])
        pltpu.sync_copy(
            g_hbm.at[pl.ds(sid * rpt, rpt)], g_sp.at[pl.ds(sid * rpt, rpt)]
        )
        plsc.subcore_barrier()

        def _fire_gathers(n, bank):
            for i in range(G):
                j = n * G + i
                pltpu.async_copy(g_sp.at[srcv.at[j]], rows.at[bank, i], gsem)

        def _fire_scatters(n, bank):
            for i in range(G):
                j = n * G + i
                pltpu.async_copy(
                    rows.at[bank, i], acc.at[dstv.at[j]], ssem, add=True
                )

        def _drain(sem, k):
            for _ in range(k):
                pltpu.make_async_copy(
                    g_hbm.at[srcv.at[0]], rows.at[0, 0], sem
                ).wait()

        for n in range(2):
            _fire_gathers(n, n)
            _drain(gsem, G)
            _fire_scatters(n, n)

        def body(n, carry):
            bank = n % 2
            _drain(ssem, G)
            _fire_gathers(n, bank)
            _drain(gsem, G)
            _fire_scatters(n, bank)
            return carry

        lax.fori_loop(2, NG, body, 0)
        _drain(ssem, 2 * G)
        plsc.subcore_barrier()
        pltpu.sync_copy(
            acc.at[pl.ds(sid * rpt, rpt)], out_hbm.at[cid, pl.ds(sid * rpt, rpt)]
        )

    return sc2


def _tc_prep(N, IN, OUT):
    def body(x_ref, w_ref, degp_ref, g_ref, dinv_ref):
        h = lax.dot_general(
            x_ref[...], w_ref[...], (((1,), (1,)), ((), ())),
            preferred_element_type=jnp.float32,
        )
        deg = jnp.sum(degp_ref[:, :N], axis=0) + 1.0
        dinv = lax.rsqrt(deg)
        g_ref[...] = h * dinv[:, None]
        dinv_ref[...] = dinv

    return pl.pallas_call(
        body,
        out_shape=[
            jax.ShapeDtypeStruct((N, OUT), jnp.float32),
            jax.ShapeDtypeStruct((N,), jnp.float32),
        ],
    )


def _tc_final(N, OUT):
    def body(accp_ref, g_ref, dinv_ref, b_ref, o_ref):
        s = jnp.sum(accp_ref[:, :N, :], axis=0) + g_ref[...]
        o_ref[...] = jnp.maximum(
            s * dinv_ref[...][:, None] + b_ref[...][None, :], 0.0
        )

    return pl.pallas_call(
        body, out_shape=jax.ShapeDtypeStruct((N, OUT), jnp.float32)
    )


def kernel(x, edge_index, W, b):
    N, IN = x.shape
    OUT = W.shape[0]
    E = edge_index.shape[1]

    G = 8
    NJ = _ceil_to(-(-E // (NW * CH)), 2 * G)
    EPW = NJ * CH
    Epad = NW * EPW
    Np = _ceil_to(N + 1, 8 * NS)

    padn = Epad - E
    src = jnp.concatenate(
        [edge_index[0], jnp.zeros((padn,), jnp.int32)]
    )
    dst = jnp.concatenate(
        [edge_index[1], jnp.full((padn,), N, jnp.int32)]
    )
    src3 = src.reshape(NW, NJ, CH)
    dst3 = dst.reshape(NW, NJ, CH)
    dst2 = dst.reshape(NW, EPW)

    degP = _deg_kernel(N, Np, EPW)(dst2)
    g, dinv = _tc_prep(N, IN, OUT)(x, W, degP)
    accP = _scatter_kernel(N, OUT, Np, NJ, G)(g, src3, dst3)
    return _tc_final(N, OUT)(accP, g, dinv, b)
